# Initial kernel scaffold; baseline (speedup 1.0000x reference)
import jax
import jax.numpy as jnp
from jax.experimental import pallas as pl


def kernel(x, embedding):
    raise NotImplementedError("write your pallas kernel here")


# speedup vs baseline: 4.2711x; 4.2711x over previous
import functools

import jax
import jax.numpy as jnp
from jax import lax
from jax.experimental import pallas as pl
from jax.experimental.pallas import tpu as pltpu
from jax.experimental.pallas import tpu_sc as plsc

NC = 2
NS = 16
NW = NC * NS

CHUNK = 128
NBUF = 8
LEAD = 4


@functools.lru_cache(maxsize=None)
def _build(n_rows: int, d: int):
    b_per_w = n_rows // NW
    n_chunks = b_per_w // CHUNK
    assert n_rows % NW == 0 and b_per_w % CHUNK == 0
    n_steady = n_chunks - 2 * LEAD
    assert n_steady % NBUF == 0 and n_chunks >= 2 * LEAD

    mesh = plsc.VectorSubcoreMesh(core_axis_name="c", subcore_axis_name="s")

    @functools.partial(
        pl.kernel,
        mesh=mesh,
        compiler_params=pltpu.CompilerParams(use_tc_tiling_on_sc=False),
        out_type=jax.ShapeDtypeStruct((n_rows, d), jnp.float32),
        scratch_types=[
            pltpu.VMEM((n_chunks, CHUNK), jnp.int32),
            pltpu.VMEM((NBUF, CHUNK, d), jnp.float32),
            pltpu.SemaphoreType.DMA,
            pltpu.SemaphoreType.DMA,
        ],
    )
    def gather_kernel(idx_hbm, table_hbm, out_hbm, idx_v, stage, gsem, wsem):
        wid = lax.axis_index("s") * NC + lax.axis_index("c")
        base = wid * b_per_w

        pltpu.sync_copy(idx_hbm.at[wid], idx_v)

        def fire_gather(c, b):
            pltpu.async_copy(table_hbm.at[idx_v.at[c]], stage.at[b], gsem)

        def drain_gather(b):
            pltpu.make_async_copy(
                out_hbm.at[pl.ds(0, CHUNK)], stage.at[b], gsem).wait()

        def fire_write(c, b):
            pltpu.async_copy(
                stage.at[b], out_hbm.at[pl.ds(base + c * CHUNK, CHUNK)], wsem)

        def drain_write(b):
            pltpu.make_async_copy(
                stage.at[b], out_hbm.at[pl.ds(0, CHUNK)], wsem).wait()

        for c in range(LEAD):
            fire_gather(c, c % NBUF)
        for c in range(LEAD):
            drain_gather(c % NBUF)
            fire_write(c, c % NBUF)
            fire_gather(c + LEAD, (c + LEAD) % NBUF)

        def body(i, carry):
            c0 = LEAD + i * NBUF
            for b in range(NBUF):
                bb = (LEAD + b) % NBUF
                c = c0 + b
                drain_gather(bb)
                fire_write(c, bb)
                drain_write((bb + LEAD) % NBUF)
                fire_gather(c + LEAD, (bb + LEAD) % NBUF)
            return carry

        lax.fori_loop(0, n_steady // NBUF, body, 0, unroll=False)

        for c in range(n_chunks - LEAD, n_chunks):
            bb = c % NBUF
            drain_gather(bb)
            fire_write(c, bb)
            drain_write((bb + LEAD) % NBUF)
        for c in range(n_chunks - LEAD, n_chunks):
            drain_write(c % NBUF)

    return gather_kernel


def kernel(x, embedding):
    n_rows = x.size
    d = embedding.shape[1]
    xr = x.reshape(-1).astype(jnp.int32)
    xr = xr.reshape(NW, n_rows // (NW * CHUNK), CHUNK)
    out = _build(n_rows, d)(xr, embedding)
    return out.reshape(*x.shape, d)


# speedup vs baseline: 4.2811x; 1.0023x over previous
import functools

import jax
import jax.numpy as jnp
from jax import lax
from jax.experimental import pallas as pl
from jax.experimental.pallas import tpu as pltpu
from jax.experimental.pallas import tpu_sc as plsc

NC = 2
NS = 16
NW = NC * NS

CHUNK = 256
NBUF = 4
LEAD = 2


@functools.lru_cache(maxsize=None)
def _build(n_rows: int, d: int):
    b_per_w = n_rows // NW
    n_chunks = b_per_w // CHUNK
    assert n_rows % NW == 0 and b_per_w % CHUNK == 0
    n_steady = n_chunks - 2 * LEAD
    assert n_steady % NBUF == 0 and n_chunks >= 2 * LEAD

    mesh = plsc.VectorSubcoreMesh(core_axis_name="c", subcore_axis_name="s")

    @functools.partial(
        pl.kernel,
        mesh=mesh,
        compiler_params=pltpu.CompilerParams(use_tc_tiling_on_sc=False),
        out_type=jax.ShapeDtypeStruct((n_rows, d), jnp.float32),
        scratch_types=[
            pltpu.VMEM((n_chunks, CHUNK), jnp.int32),
            pltpu.VMEM((NBUF, CHUNK, d), jnp.float32),
            pltpu.SemaphoreType.DMA,
            pltpu.SemaphoreType.DMA,
        ],
    )
    def gather_kernel(idx_hbm, table_hbm, out_hbm, idx_v, stage, gsem, wsem):
        wid = lax.axis_index("s") * NC + lax.axis_index("c")
        base = wid * b_per_w

        pltpu.sync_copy(idx_hbm.at[wid], idx_v)

        def fire_gather(c, b):
            pltpu.async_copy(table_hbm.at[idx_v.at[c]], stage.at[b], gsem)

        def drain_gather(b):
            pltpu.make_async_copy(
                out_hbm.at[pl.ds(0, CHUNK)], stage.at[b], gsem).wait()

        def fire_write(c, b):
            pltpu.async_copy(
                stage.at[b], out_hbm.at[pl.ds(base + c * CHUNK, CHUNK)], wsem)

        def drain_write(b):
            pltpu.make_async_copy(
                stage.at[b], out_hbm.at[pl.ds(0, CHUNK)], wsem).wait()

        for c in range(LEAD):
            fire_gather(c, c % NBUF)
        for c in range(LEAD):
            drain_gather(c % NBUF)
            fire_write(c, c % NBUF)
            fire_gather(c + LEAD, (c + LEAD) % NBUF)

        def body(i, carry):
            c0 = LEAD + i * NBUF
            for b in range(NBUF):
                bb = (LEAD + b) % NBUF
                c = c0 + b
                drain_gather(bb)
                fire_write(c, bb)
                drain_write((bb + LEAD) % NBUF)
                fire_gather(c + LEAD, (bb + LEAD) % NBUF)
            return carry

        lax.fori_loop(0, n_steady // NBUF, body, 0, unroll=False)

        for c in range(n_chunks - LEAD, n_chunks):
            bb = c % NBUF
            drain_gather(bb)
            fire_write(c, bb)
            drain_write((bb + LEAD) % NBUF)
        for c in range(n_chunks - LEAD, n_chunks):
            drain_write(c % NBUF)

    return gather_kernel


def kernel(x, embedding):
    n_rows = x.size
    d = embedding.shape[1]
    xr = x.reshape(-1).astype(jnp.int32)
    xr = xr.reshape(NW, n_rows // (NW * CHUNK), CHUNK)
    out = _build(n_rows, d)(xr, embedding)
    return out.reshape(*x.shape, d)
